# baseline (speedup 1.0000x reference)
import functools

import jax
import jax.numpy as jnp
from jax import lax
from jax.experimental import pallas as pl
from jax.experimental.pallas import tpu as pltpu
from jax.experimental.pallas import tpu_sc as plsc

_NUM_CATEGORIES = 1000
_ROWS = 1024 * 26
_NC, _NS, _L = 2, 16, 16
_NW = _NC * _NS
_ROWS_PER_W = _ROWS // _NW
_CHUNK = 32
_NCHUNK = _ROWS_PER_W // _CHUNK


def _body(idx_hbm, out_hbm, idx_v, buf0, buf1, sem0, sem1):
    wid = lax.axis_index("s") * _NC + lax.axis_index("c")
    row_base = wid * _ROWS_PER_W

    pltpu.sync_copy(idx_hbm.at[pl.ds(row_base, _ROWS_PER_W)], idx_v)

    zeros = jnp.zeros((_L,), jnp.int32)
    ones = jnp.full((_L,), 1, jnp.int32)
    iota = lax.iota(jnp.int32, _L)

    offs = [o * _L for o in range(_NUM_CATEGORIES // _L)]
    offs.append(_NUM_CATEGORIES - _L)

    def _zero(r, carry):
        for off in offs:
            buf0[r, pl.ds(off, _L)] = zeros
            buf1[r, pl.ds(off, _L)] = zeros
        return carry

    lax.fori_loop(0, _CHUNK, _zero, 0)

    bufs = (buf0, buf1)
    sems = (sem0, sem1)
    copies = [None, None]
    groups = _CHUNK // _L

    def _rc_idx(chunk, o):
        cols = idx_v[pl.ds(chunk * _CHUNK + o * _L, _L)]
        rows = o * _L + iota
        mask = (cols >= 0) & (cols < _NUM_CATEGORIES)
        return rows, cols, mask

    for i in range(_NCHUNK):
        b = i & 1
        buf = bufs[b]
        if i >= 2:
            copies[b].wait()
            for o in range(groups):
                rows, cols, mask = _rc_idx(i - 2, o)
                plsc.store_scatter(buf, [rows, cols], zeros, mask=mask)
        for o in range(groups):
            rows, cols, mask = _rc_idx(i, o)
            plsc.store_scatter(buf, [rows, cols], ones, mask=mask)
        dst = out_hbm.at[pl.ds(row_base + i * _CHUNK, _CHUNK)]
        copies[b] = pltpu.async_copy(buf, dst, sems[b])

    copies[0].wait()
    copies[1].wait()


_onehot_sc = functools.partial(
    pl.kernel,
    out_type=jax.ShapeDtypeStruct((_ROWS, _NUM_CATEGORIES), jnp.int32),
    mesh=plsc.VectorSubcoreMesh(core_axis_name="c", subcore_axis_name="s"),
    compiler_params=pltpu.CompilerParams(needs_layout_passes=False),
    scratch_types=[
        pltpu.VMEM((_ROWS_PER_W,), jnp.int32),
        pltpu.VMEM((_CHUNK, _NUM_CATEGORIES), jnp.int32),
        pltpu.VMEM((_CHUNK, _NUM_CATEGORIES), jnp.int32),
        pltpu.SemaphoreType.DMA,
        pltpu.SemaphoreType.DMA,
    ],
)(_body)


@jax.jit
def kernel(tensor):
    idx = tensor.reshape(-1).astype(jnp.int32)
    flat = _onehot_sc(idx)
    return flat.reshape(tensor.shape + (_NUM_CATEGORIES,))
